# baseline (speedup 1.0000x reference)
import jax
import jax.numpy as jnp
import numpy as np
from jax.experimental import pallas as pl
from jax.experimental.pallas import tpu as pltpu

_R = 2048
_C = 1024
_CHUNK = 128


def _cumsum_tile_kernel(x_ref, u_ref, o_ref, carry_ref):
    j = pl.program_id(1)

    @pl.when(j == 0)
    def _init():
        carry_ref[...] = jnp.zeros_like(carry_ref)

    xb = x_ref[...].astype(jnp.bfloat16)
    u = u_ref[...]
    carry = carry_ref[...]
    for k in range(_C // _CHUNK):
        y = jnp.dot(xb[:, k * _CHUNK:(k + 1) * _CHUNK], u,
                    preferred_element_type=jnp.float32) + carry
        o_ref[:, k * _CHUNK:(k + 1) * _CHUNK] = y
        carry = jnp.broadcast_to(y[:, _CHUNK - 1:_CHUNK], carry.shape)
    carry_ref[...] = carry


def kernel(x):
    x = x.astype(jnp.float32)
    n, m = x.shape
    u = jnp.asarray(np.triu(np.ones((_CHUNK, _CHUNK), dtype=np.float32)),
                    dtype=jnp.bfloat16)
    grid = (n // _R, m // _C)
    return pl.pallas_call(
        _cumsum_tile_kernel,
        grid=grid,
        in_specs=[
            pl.BlockSpec((_R, _C), lambda i, j: (i, j)),
            pl.BlockSpec((_CHUNK, _CHUNK), lambda i, j: (0, 0)),
        ],
        out_specs=pl.BlockSpec((_R, _C), lambda i, j: (i, j)),
        out_shape=jax.ShapeDtypeStruct((n, m), jnp.float32),
        scratch_shapes=[pltpu.VMEM((_R, _CHUNK), jnp.float32)],
        compiler_params=pltpu.CompilerParams(
            dimension_semantics=("parallel", "arbitrary")),
    )(x, u)
